# Initial kernel scaffold; baseline (speedup 1.0000x reference)
import jax
import jax.numpy as jnp
from jax.experimental import pallas as pl


def kernel(x, edge_index, W):
    raise NotImplementedError("write your pallas kernel here")


# speedup vs baseline: 7.7327x; 7.7327x over previous
import functools

import jax
import jax.numpy as jnp
from jax import lax
from jax.experimental import pallas as pl
from jax.experimental.pallas import tpu as pltpu
from jax.experimental.pallas import tpu_sc as plsc

N = 10000
D = 128
E = 320000
NC = 2
NS = 16
CH = 80
NCHUNK = (E // (NC * NS)) // CH
ROWS_A = 624
TAIL_BASE = ROWS_A * NS
TAIL = N - TAIL_BASE
ZR = 48


def _sc_body(src_hbm, dst_hbm, x_hbm, out_hbm,
             src_v, dst_v, rows_v, stage_v, agg_sh, sem):
    cid = lax.axis_index("c")
    sid = lax.axis_index("s")

    z16 = jnp.zeros((16,), jnp.float32)

    @pl.loop(0, ZR)
    def _zero(i):
        for l in range(D // 16):
            stage_v[i, pl.ds(l * 16, 16)] = z16

    base = sid * ROWS_A

    @pl.loop(0, ROWS_A // ZR)
    def _zero_dma(i):
        pltpu.sync_copy(stage_v, agg_sh.at[pl.ds(base + i * ZR, ZR)])

    @pl.when(sid == NS - 1)
    def _zero_tail():
        pltpu.sync_copy(stage_v.at[pl.ds(0, TAIL)],
                        agg_sh.at[pl.ds(TAIL_BASE, TAIL)])

    pltpu.sync_copy(src_hbm.at[cid, sid], src_v)
    pltpu.sync_copy(dst_hbm.at[cid, sid], dst_v)

    plsc.subcore_barrier()

    @pl.loop(0, NCHUNK)
    def _edges(j):
        pltpu.async_copy(x_hbm.at[src_v.at[j]], rows_v, sem).wait()
        pltpu.sync_copy(rows_v, agg_sh.at[dst_v.at[j]], add=True)

    plsc.subcore_barrier()

    sl = pl.ds(base, ROWS_A)
    pltpu.sync_copy(agg_sh.at[sl], out_hbm.at[cid].at[sl])

    @pl.when(sid == NS - 1)
    def _write_tail():
        tl = pl.ds(TAIL_BASE, TAIL)
        pltpu.sync_copy(agg_sh.at[tl], out_hbm.at[cid].at[tl])


_sc_scatter = functools.partial(
    pl.kernel,
    out_type=jax.ShapeDtypeStruct((NC, N, D), jnp.float32),
    mesh=plsc.VectorSubcoreMesh(core_axis_name="c", subcore_axis_name="s"),
    scratch_types=[
        pltpu.VMEM((NCHUNK, CH), jnp.int32),
        pltpu.VMEM((NCHUNK, CH), jnp.int32),
        pltpu.VMEM((CH, D), jnp.float32),
        pltpu.VMEM((ZR, D), jnp.float32),
        pltpu.VMEM_SHARED((N, D), jnp.float32),
        pltpu.SemaphoreType.DMA,
    ],
)(_sc_body)


MM_BLK = 1000


def _mm_body(p_ref, w_ref, o_ref):
    acc = p_ref[0] + p_ref[1]
    o_ref[...] = lax.dot_general(
        acc, w_ref[...], (((1,), (1,)), ((), ())),
        preferred_element_type=jnp.float32)


def _tc_matmul(partials, W):
    return pl.pallas_call(
        _mm_body,
        grid=(N // MM_BLK,),
        in_specs=[
            pl.BlockSpec((NC, MM_BLK, D), lambda i: (0, i, 0)),
            pl.BlockSpec((D, D), lambda i: (0, 0)),
        ],
        out_specs=pl.BlockSpec((MM_BLK, D), lambda i: (i, 0)),
        out_shape=jax.ShapeDtypeStruct((N, D), jnp.float32),
    )(partials, W)


@jax.jit
def kernel(x, edge_index, W):
    src = edge_index[0].astype(jnp.int32).reshape(NC, NS, NCHUNK, CH)
    dst = edge_index[1].astype(jnp.int32).reshape(NC, NS, NCHUNK, CH)
    partials = _sc_scatter(src, dst, x)
    return _tc_matmul(partials, W)
